# baseline (speedup 1.0000x reference)
import functools

import jax
import jax.numpy as jnp
from jax import lax
from jax.experimental import pallas as pl
from jax.experimental.pallas import tpu as pltpu
from jax.experimental.pallas import tpu_sc as plsc

DIM = 16
BATCH = 16384
NUM_CORES = 2
NUM_SUBCORES = 16
LANES = 16
NUM_WORKERS = NUM_CORES * NUM_SUBCORES
BPW = BATCH // NUM_WORKERS
IDX_CHUNK = 128
N_CHUNKS = BPW // IDX_CHUNK


@functools.partial(
    pl.kernel,
    out_type=jax.ShapeDtypeStruct((BATCH,), jnp.float32),
    mesh=plsc.VectorSubcoreMesh(core_axis_name="c", subcore_axis_name="s"),
    compiler_params=pltpu.CompilerParams(use_tc_tiling_on_sc=False),
    scratch_types=[
        pltpu.VMEM((N_CHUNKS, IDX_CHUNK), jnp.int32),
        pltpu.VMEM((N_CHUNKS, IDX_CHUNK), jnp.int32),
        pltpu.VMEM((BPW, DIM), jnp.float32),
        pltpu.VMEM((BPW, DIM), jnp.float32),
        pltpu.VMEM((BPW,), jnp.float32),
        pltpu.SemaphoreType.DMA,
        pltpu.SemaphoreType.DMA,
    ],
)
def _mf_kernel(idx0_hbm, idx1_hbm, u_hbm, v_hbm, out_hbm,
               idx0_v, idx1_v, urows, vrows, outv, sem_u, sem_v):
    wid = lax.axis_index("s") * NUM_CORES + lax.axis_index("c")
    base = wid * BPW

    for j in range(N_CHUNKS):
        pltpu.sync_copy(idx0_hbm.at[pl.ds(base + j * IDX_CHUNK, IDX_CHUNK)],
                        idx0_v.at[j])
        pltpu.sync_copy(idx1_hbm.at[pl.ds(base + j * IDX_CHUNK, IDX_CHUNK)],
                        idx1_v.at[j])

    copies = []
    for j in range(N_CHUNKS):
        copies.append(pltpu.async_copy(
            u_hbm.at[idx0_v.at[j]],
            urows.at[pl.ds(j * IDX_CHUNK, IDX_CHUNK)], sem_u))
        copies.append(pltpu.async_copy(
            v_hbm.at[idx1_v.at[j]],
            vrows.at[pl.ds(j * IDX_CHUNK, IDX_CHUNK)], sem_v))
    for c in copies:
        c.wait()

    lanes = lax.iota(jnp.int32, LANES)
    perm8 = lanes ^ 8
    perm4 = lanes ^ 4
    perm2 = lanes ^ 2
    perm1 = lanes ^ 1

    def shuf(x, perm):
        return x.at[perm].get(mode="promise_in_bounds", unique_indices=True)

    def group(g, carry):
        gbase = pl.multiple_of(g * LANES, LANES)
        acc = jnp.zeros((LANES,), jnp.float32)
        for r in range(LANES):
            p = urows[gbase + r, :] * vrows[gbase + r, :]
            p = p + shuf(p, perm8)
            p = p + shuf(p, perm4)
            p = p + shuf(p, perm2)
            p = p + shuf(p, perm1)
            acc = jnp.where(lanes == r, p, acc)
        outv[pl.ds(gbase, LANES)] = acc
        return carry

    lax.fori_loop(0, BPW // LANES, group, 0)

    pltpu.sync_copy(outv, out_hbm.at[pl.ds(base, BPW)])


def kernel(x, U, V):
    xi = x.astype(jnp.int32)
    return _mf_kernel(xi[:, 0], xi[:, 1], U, V)
